# baseline (speedup 1.0000x reference)
import functools
import jax
import jax.numpy as jnp
from jax import lax
from jax.experimental import pallas as pl
from jax.experimental.pallas import tpu as pltpu
from jax.experimental.pallas import tpu_sc as plsc

K = 10
R_BLK = 200
C = 128
NSC = 2
NTILE = 16


def _pad_rows(n):
    return ((n + 1 + NTILE * 8 - 1) // (NTILE * 8)) * (NTILE * 8)


def _sc_hop(g, srcp, dstp, zeros, nc, acc_rows, d):
    rpt = acc_rows // NTILE

    mesh = plsc.VectorSubcoreMesh(core_axis_name="c", subcore_axis_name="s")

    @functools.partial(
        pl.kernel,
        mesh=mesh,
        out_type=jax.ShapeDtypeStruct((NSC, acc_rows, d), jnp.float32),
        scratch_types=[
            pltpu.VMEM((C,), jnp.int32),
            pltpu.VMEM((C,), jnp.int32),
            pltpu.VMEM((C,), jnp.int32),
            pltpu.VMEM((C,), jnp.int32),
            pltpu.VMEM((C, d), jnp.float32),
            pltpu.VMEM((C, d), jnp.float32),
            pltpu.VMEM_SHARED((acc_rows, d), jnp.float32),
            pltpu.SemaphoreType.DMA,
            pltpu.SemaphoreType.DMA,
        ],
    )
    def hop(g_hbm, src_hbm, dst_hbm, z_hbm, out_hbm, src_a, src_b, dst_a,
            dst_b, rows0, rows1, acc, sem0, sem1):
        c = lax.axis_index("c")
        s = lax.axis_index("s")
        row0 = s * rpt
        pltpu.sync_copy(z_hbm.at[pl.ds(row0, rpt)], acc.at[pl.ds(row0, rpt)])
        chunk0 = (c * NTILE + s) * nc
        plsc.subcore_barrier()

        def body(j, carry):
            ch = chunk0 + j
            pltpu.sync_copy(src_hbm.at[pl.ds(ch * C, C)], src_a)
            pltpu.sync_copy(dst_hbm.at[ch], dst_a)
            pltpu.async_copy(g_hbm.at[src_a], rows0, sem0).wait()
            pltpu.sync_copy(rows0, acc.at[dst_a], add=True)
            return carry

        lax.fori_loop(0, nc, body, 0)
        plsc.subcore_barrier()
        pltpu.sync_copy(acc.at[pl.ds(row0, rpt)],
                        out_hbm.at[c, pl.ds(row0, rpt)])

    return hop(g, srcp, dstp, zeros)


def _prep_body(pd_ref, x_ref, g0_ref, di_ref, sd_ref):
    dsum = pd_ref[0, :, 0:1] + pd_ref[1, :, 0:1] + 1.0
    di_ref[...] = 1.0 / dsum
    sd = jnp.sqrt(dsum)
    sd_ref[...] = sd
    g0_ref[...] = x_ref[...] * (sd / dsum)


def _tc_prep(p_deg, x):
    n, d = x.shape
    grid = (n // R_BLK,)
    return pl.pallas_call(
        _prep_body,
        grid=grid,
        in_specs=[
            pl.BlockSpec((NSC, R_BLK, d), lambda i: (0, i, 0)),
            pl.BlockSpec((R_BLK, d), lambda i: (i, 0)),
        ],
        out_specs=[
            pl.BlockSpec((R_BLK, d), lambda i: (i, 0)),
            pl.BlockSpec((R_BLK, 1), lambda i: (i, 0)),
            pl.BlockSpec((R_BLK, 1), lambda i: (i, 0)),
        ],
        out_shape=[
            jax.ShapeDtypeStruct((n, d), jnp.float32),
            jax.ShapeDtypeStruct((n, 1), jnp.float32),
            jax.ShapeDtypeStruct((n, 1), jnp.float32),
        ],
    )(p_deg, x)


def _comb_body(p_ref, g_ref, di_ref, o_ref):
    o_ref[...] = (p_ref[0] + p_ref[1] + g_ref[...]) * di_ref[...]


def _tc_combine(p, g, dinv2):
    n, d = g.shape
    grid = (n // R_BLK,)
    return pl.pallas_call(
        _comb_body,
        grid=grid,
        in_specs=[
            pl.BlockSpec((NSC, R_BLK, d), lambda i: (0, i, 0)),
            pl.BlockSpec((R_BLK, d), lambda i: (i, 0)),
            pl.BlockSpec((R_BLK, 1), lambda i: (i, 0)),
        ],
        out_specs=pl.BlockSpec((R_BLK, d), lambda i: (i, 0)),
        out_shape=jax.ShapeDtypeStruct((n, d), jnp.float32),
    )(p, g, dinv2)


def _final_body(g_ref, sd_ref, w_ref, b_ref, o_ref):
    G = g_ref[...]
    sd = sd_ref[...]
    kk, r, d = G.shape
    t = jax.lax.dot_general(G.reshape(kk * r, d), w_ref[...],
                            (((1,), (0,)), ((), ())))
    t = t.reshape(kk, r, 1)
    s = jax.nn.sigmoid(t * sd[None, :, :] + b_ref[0, 0])
    o_ref[...] = jnp.sum(s * G, axis=0) * sd


def _tc_final(Gstack, sqrtdeg, W, b):
    kk, n, d = Gstack.shape
    grid = (n // R_BLK,)
    return pl.pallas_call(
        _final_body,
        grid=grid,
        in_specs=[
            pl.BlockSpec((kk, R_BLK, d), lambda i: (0, i, 0)),
            pl.BlockSpec((R_BLK, 1), lambda i: (i, 0)),
            pl.BlockSpec((d, 1), lambda i: (0, 0)),
            pl.BlockSpec((1, 1), lambda i: (0, 0)),
        ],
        out_specs=pl.BlockSpec((R_BLK, d), lambda i: (i, 0)),
        out_shape=jax.ShapeDtypeStruct((n, d), jnp.float32),
    )(Gstack, sqrtdeg, W, b)


def kernel(x, edge_index, W, b):
    n, d = x.shape
    src = edge_index[0].astype(jnp.int32)
    dst = edge_index[1].astype(jnp.int32)
    e = src.shape[0]

    acc_rows = _pad_rows(n)
    trash = n

    nc = -(-e // (NSC * NTILE * C))
    nc = ((nc + 7) // 8) * 8
    ep = NSC * NTILE * C * nc
    srcp = jnp.concatenate([src, jnp.zeros((ep - e,), jnp.int32)])
    dstp = jnp.concatenate([dst, jnp.full((ep - e,), trash, jnp.int32)])
    dstp = dstp.reshape(NSC * NTILE * nc, C)

    zeros = jnp.zeros((acc_rows, d), jnp.float32)
    ones_nd = jnp.ones((n, d), jnp.float32)

    p_deg = _sc_hop(ones_nd, srcp, dstp, zeros, nc, acc_rows, d)
    g, dinv2, sqrtdeg = _tc_prep(p_deg, x)

    gs = [g]
    for _ in range(K):
        p = _sc_hop(g, srcp, dstp, zeros, nc, acc_rows, d)
        g = _tc_combine(p, g, dinv2)
        gs.append(g)
    Gstack = jnp.stack(gs, axis=0)
    return _tc_final(Gstack, sqrtdeg, W, b.reshape(1, 1))


# speedup vs baseline: 1.4144x; 1.4144x over previous
import functools
import jax
import jax.numpy as jnp
from jax import lax
from jax.experimental import pallas as pl
from jax.experimental.pallas import tpu as pltpu
from jax.experimental.pallas import tpu_sc as plsc

K = 10
R_BLK = 200
C = 128
NSC = 2
NTILE = 16


def _pad_rows(n):
    return ((n + 1 + NTILE * 8 - 1) // (NTILE * 8)) * (NTILE * 8)


def _sc_hop(g, srcp, dstp, zeros, nc, acc_rows, d):
    rpt = acc_rows // NTILE

    mesh = plsc.VectorSubcoreMesh(core_axis_name="c", subcore_axis_name="s")

    @functools.partial(
        pl.kernel,
        mesh=mesh,
        out_type=jax.ShapeDtypeStruct((NSC, acc_rows, d), jnp.float32),
        scratch_types=[
            pltpu.VMEM((C,), jnp.int32),
            pltpu.VMEM((C,), jnp.int32),
            pltpu.VMEM((C, d), jnp.float32),
            pltpu.VMEM_SHARED((acc_rows, d), jnp.float32),
            pltpu.SemaphoreType.DMA,
        ],
    )
    def hop(g_hbm, src_hbm, dst_hbm, z_hbm, out_hbm, src_a, dst_a, rows0,
            acc, sem0):
        c = lax.axis_index("c")
        s = lax.axis_index("s")
        row0 = s * rpt
        pltpu.sync_copy(z_hbm.at[pl.ds(row0, rpt)], acc.at[pl.ds(row0, rpt)])
        chunk0 = (c * NTILE + s) * nc
        plsc.subcore_barrier()

        def body(j, carry):
            ch = chunk0 + j
            pltpu.sync_copy(src_hbm.at[pl.ds(ch * C, C)], src_a)
            pltpu.sync_copy(dst_hbm.at[ch], dst_a)
            pltpu.async_copy(g_hbm.at[src_a], rows0, sem0).wait()
            pltpu.sync_copy(rows0, acc.at[dst_a], add=True)
            return carry

        lax.fori_loop(0, nc, body, 0)
        plsc.subcore_barrier()
        pltpu.sync_copy(acc.at[pl.ds(row0, rpt)],
                        out_hbm.at[c, pl.ds(row0, rpt)])

    return hop(g, srcp, dstp, zeros)


def _prep_body(pd_ref, x_ref, g0_ref, di_ref, sd_ref):
    dsum = pd_ref[0, :, 0:1] + pd_ref[1, :, 0:1] + 1.0
    di_ref[...] = 1.0 / dsum
    sd = jnp.sqrt(dsum)
    sd_ref[...] = sd
    g0_ref[...] = x_ref[...] * (sd / dsum)


def _tc_prep(p_deg, x):
    n, d = x.shape
    grid = (n // R_BLK,)
    return pl.pallas_call(
        _prep_body,
        grid=grid,
        in_specs=[
            pl.BlockSpec((NSC, R_BLK, d), lambda i: (0, i, 0)),
            pl.BlockSpec((R_BLK, d), lambda i: (i, 0)),
        ],
        out_specs=[
            pl.BlockSpec((R_BLK, d), lambda i: (i, 0)),
            pl.BlockSpec((R_BLK, 1), lambda i: (i, 0)),
            pl.BlockSpec((R_BLK, 1), lambda i: (i, 0)),
        ],
        out_shape=[
            jax.ShapeDtypeStruct((n, d), jnp.float32),
            jax.ShapeDtypeStruct((n, 1), jnp.float32),
            jax.ShapeDtypeStruct((n, 1), jnp.float32),
        ],
    )(p_deg, x)


def _comb_body(p_ref, g_ref, di_ref, o_ref):
    o_ref[...] = (p_ref[0] + p_ref[1] + g_ref[...]) * di_ref[...]


def _tc_combine(p, g, dinv2):
    n, d = g.shape
    grid = (n // R_BLK,)
    return pl.pallas_call(
        _comb_body,
        grid=grid,
        in_specs=[
            pl.BlockSpec((NSC, R_BLK, d), lambda i: (0, i, 0)),
            pl.BlockSpec((R_BLK, d), lambda i: (i, 0)),
            pl.BlockSpec((R_BLK, 1), lambda i: (i, 0)),
        ],
        out_specs=pl.BlockSpec((R_BLK, d), lambda i: (i, 0)),
        out_shape=jax.ShapeDtypeStruct((n, d), jnp.float32),
    )(p, g, dinv2)


def _final_body(g_ref, sd_ref, w_ref, b_ref, o_ref):
    G = g_ref[...]
    sd = sd_ref[...]
    kk, r, d = G.shape
    t = jax.lax.dot_general(G.reshape(kk * r, d), w_ref[...],
                            (((1,), (0,)), ((), ())))
    t = t.reshape(kk, r, 1)
    s = jax.nn.sigmoid(t * sd[None, :, :] + b_ref[0, 0])
    o_ref[...] = jnp.sum(s * G, axis=0) * sd


def _tc_final(Gstack, sqrtdeg, W, b):
    kk, n, d = Gstack.shape
    grid = (n // R_BLK,)
    return pl.pallas_call(
        _final_body,
        grid=grid,
        in_specs=[
            pl.BlockSpec((kk, R_BLK, d), lambda i: (0, i, 0)),
            pl.BlockSpec((R_BLK, 1), lambda i: (i, 0)),
            pl.BlockSpec((d, 1), lambda i: (0, 0)),
            pl.BlockSpec((1, 1), lambda i: (0, 0)),
        ],
        out_specs=pl.BlockSpec((R_BLK, d), lambda i: (i, 0)),
        out_shape=jax.ShapeDtypeStruct((n, d), jnp.float32),
    )(Gstack, sqrtdeg, W, b)


def kernel(x, edge_index, W, b):
    n, d = x.shape
    src = edge_index[0].astype(jnp.int32)
    dst = edge_index[1].astype(jnp.int32)
    e = src.shape[0]

    acc_rows = _pad_rows(n)
    trash = n

    nc = -(-e // (NSC * NTILE * C))
    ep = NSC * NTILE * C * nc
    srcp = jnp.concatenate([src, jnp.zeros((ep - e,), jnp.int32)])
    dstp = jnp.concatenate([dst, jnp.full((ep - e,), trash, jnp.int32)])
    dstp = dstp.reshape(NSC * NTILE * nc, C)

    zeros = jnp.zeros((acc_rows, d), jnp.float32)
    ones_nd = jnp.ones((n, d), jnp.float32)

    p_deg = _sc_hop(ones_nd, srcp, dstp, zeros, nc, acc_rows, d)
    g, dinv2, sqrtdeg = _tc_prep(p_deg, x)

    gs = [g]
    for _ in range(K):
        p = _sc_hop(g, srcp, dstp, zeros, nc, acc_rows, d)
        g = _tc_combine(p, g, dinv2)
        gs.append(g)
    Gstack = jnp.stack(gs, axis=0)
    return _tc_final(Gstack, sqrtdeg, W, b.reshape(1, 1))


# speedup vs baseline: 1.4686x; 1.0383x over previous
import functools
import jax
import jax.numpy as jnp
from jax import lax
from jax.experimental import pallas as pl
from jax.experimental.pallas import tpu as pltpu
from jax.experimental.pallas import tpu_sc as plsc

K = 10
R_BLK = 200
C = 128
NSC = 2
NTILE = 16


def _pad_rows(n):
    return ((n + 1 + NTILE * 8 - 1) // (NTILE * 8)) * (NTILE * 8)


def _sc_hop(g, srcp, dstp, zeros, nc, acc_rows, d):
    rpt = acc_rows // NTILE

    mesh = plsc.VectorSubcoreMesh(core_axis_name="c", subcore_axis_name="s")

    @functools.partial(
        pl.kernel,
        mesh=mesh,
        out_type=jax.ShapeDtypeStruct((NSC, acc_rows, d), jnp.float32),
        scratch_types=[
            pltpu.VMEM((C,), jnp.int32),
            pltpu.VMEM((C,), jnp.int32),
            pltpu.VMEM((C, d), jnp.float32),
            pltpu.VMEM_SHARED((acc_rows, d), jnp.float32),
            pltpu.SemaphoreType.DMA,
        ],
    )
    def hop(g_hbm, src_hbm, dst_hbm, z_hbm, out_hbm, src_a, dst_a, rows0,
            acc, sem0):
        c = lax.axis_index("c")
        s = lax.axis_index("s")
        row0 = s * rpt
        pltpu.sync_copy(z_hbm.at[pl.ds(row0, rpt)], acc.at[pl.ds(row0, rpt)])
        chunk0 = (c * NTILE + s) * nc
        plsc.subcore_barrier()

        def body(j, carry):
            ch = chunk0 + j
            pltpu.sync_copy(src_hbm.at[pl.ds(ch * C, C)], src_a)
            pltpu.sync_copy(dst_hbm.at[ch], dst_a)
            pltpu.async_copy(g_hbm.at[src_a], rows0, sem0).wait()
            pltpu.sync_copy(rows0, acc.at[dst_a], add=True)
            return carry

        lax.fori_loop(0, nc, body, 0)
        plsc.subcore_barrier()
        pltpu.sync_copy(acc.at[pl.ds(row0, rpt)],
                        out_hbm.at[c, pl.ds(row0, rpt)])

    return hop(g, srcp, dstp, zeros)


def _prep_body(pd_ref, x_ref, g0_ref, di_ref, sd_ref):
    dsum = pd_ref[0, :, 0:1] + pd_ref[1, :, 0:1] + 1.0
    di_ref[...] = 1.0 / dsum
    sd = jnp.sqrt(dsum)
    sd_ref[...] = sd
    g0_ref[...] = x_ref[...] * (sd / dsum)


def _tc_prep(p_deg, x):
    n, d = x.shape
    grid = (n // R_BLK,)
    return pl.pallas_call(
        _prep_body,
        grid=grid,
        in_specs=[
            pl.BlockSpec((NSC, R_BLK, d), lambda i: (0, i, 0)),
            pl.BlockSpec((R_BLK, d), lambda i: (i, 0)),
        ],
        out_specs=[
            pl.BlockSpec((R_BLK, d), lambda i: (i, 0)),
            pl.BlockSpec((R_BLK, 1), lambda i: (i, 0)),
            pl.BlockSpec((R_BLK, 1), lambda i: (i, 0)),
        ],
        out_shape=[
            jax.ShapeDtypeStruct((n, d), jnp.float32),
            jax.ShapeDtypeStruct((n, 1), jnp.float32),
            jax.ShapeDtypeStruct((n, 1), jnp.float32),
        ],
    )(p_deg, x)


def _comb_body(p_ref, g_ref, di_ref, o_ref):
    o_ref[...] = (p_ref[0] + p_ref[1] + g_ref[...]) * di_ref[...]


def _tc_combine(p, g, dinv2):
    n, d = g.shape
    grid = (n // R_BLK,)
    return pl.pallas_call(
        _comb_body,
        grid=grid,
        in_specs=[
            pl.BlockSpec((NSC, R_BLK, d), lambda i: (0, i, 0)),
            pl.BlockSpec((R_BLK, d), lambda i: (i, 0)),
            pl.BlockSpec((R_BLK, 1), lambda i: (i, 0)),
        ],
        out_specs=pl.BlockSpec((R_BLK, d), lambda i: (i, 0)),
        out_shape=jax.ShapeDtypeStruct((n, d), jnp.float32),
    )(p, g, dinv2)


def _final_body(*refs):
    gl = refs[:K + 1]
    sd_ref, w_ref, b_ref, o_ref = refs[K + 1:]
    sd = sd_ref[...]
    G = jnp.stack([g[...] for g in gl], axis=0)
    kk, r, d = G.shape
    t = jax.lax.dot_general(G.reshape(kk * r, d), w_ref[...],
                            (((1,), (0,)), ((), ())))
    t = t.reshape(kk, r, 1)
    s = jax.nn.sigmoid(t * sd[None, :, :] + b_ref[0, 0])
    o_ref[...] = jnp.sum(s * G, axis=0) * sd


def _tc_final(gs, sqrtdeg, W, b):
    n, d = gs[0].shape
    grid = (n // R_BLK,)
    gspec = pl.BlockSpec((R_BLK, d), lambda i: (i, 0))
    return pl.pallas_call(
        _final_body,
        grid=grid,
        in_specs=[gspec] * (K + 1) + [
            pl.BlockSpec((R_BLK, 1), lambda i: (i, 0)),
            pl.BlockSpec((d, 1), lambda i: (0, 0)),
            pl.BlockSpec((1, 1), lambda i: (0, 0)),
        ],
        out_specs=pl.BlockSpec((R_BLK, d), lambda i: (i, 0)),
        out_shape=jax.ShapeDtypeStruct((n, d), jnp.float32),
    )(*gs, sqrtdeg, W, b)


def kernel(x, edge_index, W, b):
    n, d = x.shape
    src = edge_index[0].astype(jnp.int32)
    dst = edge_index[1].astype(jnp.int32)
    e = src.shape[0]

    acc_rows = _pad_rows(n)
    trash = n

    nc = -(-e // (NSC * NTILE * C))
    ep = NSC * NTILE * C * nc
    srcp = jnp.concatenate([src, jnp.zeros((ep - e,), jnp.int32)])
    dstp = jnp.concatenate([dst, jnp.full((ep - e,), trash, jnp.int32)])
    dstp = dstp.reshape(NSC * NTILE * nc, C)

    zeros = jnp.zeros((acc_rows, d), jnp.float32)
    ones_nd = jnp.ones((n, d), jnp.float32)

    p_deg = _sc_hop(ones_nd, srcp, dstp, zeros, nc, acc_rows, d)
    g, dinv2, sqrtdeg = _tc_prep(p_deg, x)

    gs = [g]
    for _ in range(K):
        p = _sc_hop(g, srcp, dstp, zeros, nc, acc_rows, d)
        g = _tc_combine(p, g, dinv2)
        gs.append(g)
    return _tc_final(gs, sqrtdeg, W, b.reshape(1, 1))
